# Initial kernel scaffold; baseline (speedup 1.0000x reference)
import jax
import jax.numpy as jnp
from jax.experimental import pallas as pl


def kernel(input_ids, token_type_ids, word_embeddings, position_embeddings, token_type_embeddings, ln_gamma, ln_beta):
    raise NotImplementedError("write your pallas kernel here")


# speedup vs baseline: 1.9018x; 1.9018x over previous
import functools
import jax
import jax.numpy as jnp
from jax import lax
from jax.experimental import pallas as pl
from jax.experimental.pallas import tpu as pltpu
from jax.experimental.pallas import tpu_sc as plsc

HID = 1024
B = 32
S = 512
EPS = 1e-12
NTOK = B * S
NW = 32
TOK_PER_W = NTOK // NW
CHUNK = 32
NCHUNK = TOK_PER_W // CHUNK


def _gather_body(table, ids, out, idx_v, buf0, buf1, gs0, gs1, ws0, ws1):
    wid = lax.axis_index("s") * 2 + lax.axis_index("c")
    base = wid * TOK_PER_W
    pltpu.sync_copy(ids.at[pl.ds(base, TOK_PER_W)], idx_v)

    bufs = (buf0, buf1)
    gsems = (gs0, gs1)
    wsems = (ws0, ws1)

    def start_gather(c, p):
        src = table.at[idx_v.at[pl.ds(c * CHUNK, CHUNK)]]
        return pltpu.async_copy(src, bufs[p], gsems[p])

    def start_write(c, p):
        dst = out.at[pl.ds(base + c * CHUNK, CHUNK)]
        return pltpu.async_copy(bufs[p], dst, wsems[p])

    hg = [None, None]
    hw = [None, None]
    hg[0] = start_gather(0, 0)
    for c in range(NCHUNK):
        p = c & 1
        hg[p].wait()
        if c + 1 < NCHUNK:
            if c >= 1:
                hw[1 - p].wait()
            hg[1 - p] = start_gather(c + 1, 1 - p)
        hw[p] = start_write(c, p)
    hw[(NCHUNK - 1) & 1].wait()
    hw[NCHUNK & 1].wait()


@jax.jit
def _sc_gather(table, ids_flat):
    mesh = plsc.VectorSubcoreMesh(core_axis_name="c", subcore_axis_name="s")
    f = pl.kernel(
        _gather_body,
        out_type=jax.ShapeDtypeStruct((NTOK, HID), jnp.float32),
        mesh=mesh,
        scratch_types=[
            pltpu.VMEM((TOK_PER_W,), jnp.int32),
            pltpu.VMEM((CHUNK, HID), jnp.float32),
            pltpu.VMEM((CHUNK, HID), jnp.float32),
            pltpu.SemaphoreType.DMA,
            pltpu.SemaphoreType.DMA,
            pltpu.SemaphoreType.DMA,
            pltpu.SemaphoreType.DMA,
        ],
    )
    return f(table, ids_flat)


def _ln_body(tt_ref, rows_ref, pe_ref, tte_ref, gamma_ref, beta_ref, out_ref):
    rows = rows_ref[0]
    pe = pe_ref[...]
    tte0 = tte_ref[0, :HID]
    dtte = tte_ref[0, HID:] - tte0
    tt = tt_ref[0, 0, :].astype(jnp.float32)[:, None]
    emb = rows + pe + tte0[None, :] + tt * dtte[None, :]
    mean = jnp.mean(emb, axis=1, keepdims=True)
    d = emb - mean
    var = jnp.mean(d * d, axis=1, keepdims=True)
    out = d * lax.rsqrt(var + EPS) * gamma_ref[0][None, :] + beta_ref[0][None, :]
    out_ref[0] = out


@jax.jit
def _tc_ln(tt3, rows3, pe, tte2, g2, b2):
    return pl.pallas_call(
        _ln_body,
        grid=(B,),
        in_specs=[
            pl.BlockSpec((1, 1, S), lambda i: (i, 0, 0)),
            pl.BlockSpec((1, S, HID), lambda i: (i, 0, 0)),
            pl.BlockSpec((S, HID), lambda i: (0, 0)),
            pl.BlockSpec((1, 2 * HID), lambda i: (0, 0)),
            pl.BlockSpec((1, HID), lambda i: (0, 0)),
            pl.BlockSpec((1, HID), lambda i: (0, 0)),
        ],
        out_specs=pl.BlockSpec((1, S, HID), lambda i: (i, 0, 0)),
        out_shape=jax.ShapeDtypeStruct((B, S, HID), jnp.float32),
    )(tt3, rows3, pe, tte2, g2, b2)


def kernel(input_ids, token_type_ids, word_embeddings, position_embeddings,
           token_type_embeddings, ln_gamma, ln_beta):
    ids_flat = input_ids.reshape(-1)
    rows = _sc_gather(word_embeddings, ids_flat)
    rows3 = rows.reshape(B, S, HID)
    tt3 = token_type_ids.reshape(B, 1, S)
    tte2 = token_type_embeddings.reshape(1, 2 * HID)
    g2 = ln_gamma.reshape(1, HID)
    b2 = ln_beta.reshape(1, HID)
    return _tc_ln(tt3, rows3, position_embeddings, tte2, g2, b2)
